# Initial kernel scaffold; baseline (speedup 1.0000x reference)
import jax
import jax.numpy as jnp
from jax.experimental import pallas as pl


def kernel(x, edge_index, batch, seq, global_f, params):
    raise NotImplementedError("write your pallas kernel here")


# speedup vs baseline: 4.4934x; 4.4934x over previous
import jax
import jax.numpy as jnp
from jax import lax
from jax.experimental import pallas as pl
from jax.experimental.pallas import tpu as pltpu
from jax.experimental.pallas import tpu_sc as plsc

N = 10000
E = 320000
G = 64
EMB = 128
HEADS = 4
NC = 2
NS = 16
FH = EMB // NC
CH = 80
EPT = E // NS
NCHUNK = EPT // CH
R8 = (N // NS) // 8 * 8
RREM = N - NS * R8
BN = 2000


def _leaky(v):
    return jnp.where(v > 0, v, 0.1 * v)


def _b16(v):
    return v.astype(jnp.bfloat16)


def _dot(a, b):
    return lax.dot_general(_b16(a), _b16(b), (((1,), (0,)), ((), ())),
                           preferred_element_type=jnp.float32)


def _sc_body(h_hbm, src_hbm, dst_hbm, zeros_hbm, agg_hbm,
             idx_src, idx_dst, idx_off, rows, acc, sem):
    c = lax.axis_index("c")
    s = lax.axis_index("s")
    r0 = s * R8
    pltpu.sync_copy(zeros_hbm.at[pl.ds(r0, R8)], acc.at[pl.ds(r0, R8)])

    @pl.when(s == NS - 1)
    def _zero_rem():
        pltpu.sync_copy(zeros_hbm.at[pl.ds(NS * R8, RREM)],
                        acc.at[pl.ds(NS * R8, RREM)])

    pltpu.sync_copy(src_hbm.at[s], idx_src)
    pltpu.sync_copy(dst_hbm.at[s], idx_dst)
    plsc.subcore_barrier()

    cN = c * N

    def step(j, carry):
        for i in range(CH // 16):
            idx_off[pl.ds(i * 16, 16)] = idx_src[j, pl.ds(i * 16, 16)] + cN
        pltpu.async_copy(h_hbm.at[idx_off], rows, sem).wait()
        pltpu.sync_copy(rows, acc.at[idx_dst.at[j]], add=True)
        return carry

    lax.fori_loop(0, NCHUNK, step, 0, unroll=False)
    plsc.subcore_barrier()

    pltpu.sync_copy(acc.at[pl.ds(r0, R8)], agg_hbm.at[pl.ds(cN + r0, R8)])

    @pl.when(s == NS - 1)
    def _copy_rem():
        pltpu.sync_copy(acc.at[pl.ds(NS * R8, RREM)],
                        agg_hbm.at[pl.ds(cN + NS * R8, RREM)])


def _sc_spmm(h_flat, src3, dst3, zeros):
    mesh = plsc.VectorSubcoreMesh(core_axis_name="c", subcore_axis_name="s",
                                  num_cores=NC, num_subcores=NS)
    scratch = [
        pltpu.VMEM((NCHUNK, CH), jnp.int32),
        pltpu.VMEM((NCHUNK, CH), jnp.int32),
        pltpu.VMEM((CH,), jnp.int32),
        pltpu.VMEM((CH, FH), jnp.float32),
        pltpu.VMEM_SHARED((N, FH), jnp.float32),
        pltpu.SemaphoreType.DMA,
    ]
    fn = pl.kernel(_sc_body,
                   out_type=jax.ShapeDtypeStruct((NC * N, FH), jnp.float32),
                   mesh=mesh, scratch_types=scratch,
                   compiler_params=pltpu.CompilerParams(use_tc_tiling_on_sc=False))
    return fn(h_flat, src3, dst3, zeros)


def _tc_layer_body(a0, a1, p0, p1, Wrel_r, Wroot_r, brel_r, out_ref):
    agg = jnp.concatenate([a0[...], a1[...]], axis=1)
    hp = jnp.concatenate([p0[...], p1[...]], axis=1)
    hn = _dot(agg, Wrel_r[...]) + brel_r[...] + _dot(hp, Wroot_r[...])
    out_ref[0] = hn[:, :FH]
    out_ref[1] = hn[:, FH:]


def _tc_layer(a0, a1, p0, p1, Wrel, Wroot, brel):
    def nmap(i):
        return (i, 0)

    def cmap(i):
        return (0, 0)

    half = pl.BlockSpec((BN, FH), nmap)
    return pl.pallas_call(
        _tc_layer_body,
        grid=(N // BN,),
        in_specs=[half, half, half, half,
                  pl.BlockSpec(Wrel.shape, cmap), pl.BlockSpec(Wroot.shape, cmap),
                  pl.BlockSpec((1, EMB), cmap)],
        out_specs=pl.BlockSpec((2, BN, FH), lambda i: (0, i, 0)),
        out_shape=jax.ShapeDtypeStruct((2, N, FH), jnp.float32),
        compiler_params=pltpu.CompilerParams(
            dimension_semantics=("arbitrary",)),
    )(a0, a1, p0, p1, Wrel, Wroot, brel)


def _tc_final_body(h30, h31, u_ref, seq_ref, gf_ref,
                   Wsr, bsr, Wgr, bgr,
                   Wqr, bqr, Wkr, bkr, Wvr, bvr, Wpr, bpr, scaler,
                   W1r, b1fr, W2r, b2fr,
                   out_ref, acc_ref):
    i = pl.program_id(0)

    @pl.when(i == 0)
    def _init():
        acc_ref[...] = jnp.zeros_like(acc_ref)

    ub = u_ref[...]
    dn = (((0,), (0,)), ((), ()))
    for k, r in enumerate((h30, h31)):
        acc_ref[k] += lax.dot_general(ub, r[...], dn,
                                      precision=lax.Precision.HIGHEST,
                                      preferred_element_type=jnp.float32)

    @pl.when(i == pl.num_programs(0) - 1)
    def _tail():
        g = jnp.concatenate([acc_ref[0], acc_ref[1]], axis=1)
        seq_rep = _leaky(_dot(seq_ref[...], Wsr[...]) + bsr[...])
        glob = _leaky(_dot(gf_ref[...], Wgr[...]) + bgr[...])
        toks = (g, seq_rep, glob)
        Q = [_dot(t, Wqr[...]) + bqr[...] for t in toks]
        K = [_dot(t, Wkr[...]) + bkr[...] for t in toks]
        V = [_dot(t, Wvr[...]) + bvr[...] for t in toks]
        scale = scaler[...]
        a1 = jnp.zeros((G, EMB), dtype=jnp.float32)
        for q in range(3):
            o_parts = []
            for h in range(HEADS):
                sl = slice(h * EMB, (h + 1) * EMB)
                Qb = _b16(Q[q][:, sl]).astype(jnp.float32)
                sc = [jnp.sum(Qb * _b16(K[kk][:, sl]).astype(jnp.float32),
                              axis=1, keepdims=True) / scale
                      for kk in range(3)]
                m = jnp.maximum(jnp.maximum(sc[0], sc[1]), sc[2])
                e = [jnp.exp(sv - m) for sv in sc]
                den = e[0] + e[1] + e[2]
                oh = jnp.zeros((G, EMB), dtype=jnp.float32)
                for kk in range(3):
                    ab = _b16(e[kk] / den).astype(jnp.float32)
                    vb = _b16(V[kk][:, sl]).astype(jnp.float32)
                    oh += ab * vb
                o_parts.append(oh)
            o_q = jnp.concatenate(o_parts, axis=1)
            a1 += _dot(o_q, Wpr[...]) + bpr[...]
        h1 = _leaky(_dot(a1, W1r[...]) + b1fr[...])
        out_ref[...] = _dot(h1, W2r[...]) + b2fr[...]


def _tc_final(h30, h31, u, seq, gf, pvals):
    def nmap(i):
        return (i, 0)

    def cmap(i):
        return (0, 0)

    half = pl.BlockSpec((BN, FH), nmap)
    in_specs = [half, half, pl.BlockSpec((BN, G), nmap)]
    in_specs += [pl.BlockSpec(t.shape, cmap) for t in (seq, gf)]
    in_specs += [pl.BlockSpec(t.shape, cmap) for t in pvals]
    return pl.pallas_call(
        _tc_final_body,
        grid=(N // BN,),
        in_specs=in_specs,
        out_specs=pl.BlockSpec((G, EMB), cmap),
        out_shape=jax.ShapeDtypeStruct((G, EMB), jnp.float32),
        scratch_shapes=[pltpu.VMEM((2, G, FH), jnp.float32)],
        compiler_params=pltpu.CompilerParams(
            dimension_semantics=("arbitrary",)),
    )(h30, h31, u, seq, gf, *pvals)


def kernel(x, edge_index, batch, seq, global_f, params):
    src3 = edge_index[0].reshape(NS, NCHUNK, CH)
    dst3 = edge_index[1].reshape(NS, NCHUNK, CH)
    zeros = jnp.zeros((N, FH), dtype=jnp.float32)
    u = jax.nn.one_hot(batch, G, dtype=jnp.float32)
    p = params
    row = lambda a: a.reshape(1, -1)

    h = jnp.stack([x[:, :FH], x[:, FH:]])
    for l in range(3):
        agg_flat = _sc_spmm(h.reshape(NC * N, FH), src3, dst3, zeros)
        h = _tc_layer(agg_flat[:N], agg_flat[N:], h[0], h[1],
                      p['Wrel%d' % l], p['Wroot%d' % l], row(p['brel%d' % l]))

    pvals = (
        p['Ws'], row(p['bs']), p['Wg'], row(p['bg']),
        p['Wq'], row(p['bq']), p['Wk'], row(p['bk']),
        p['Wv'], row(p['bv']), p['Wp'], row(p['bp']),
        p['scale'].reshape(1, 1),
        p['W1'], row(p['b1']),
        jnp.broadcast_to(p['W2'], (EMB, EMB)), jnp.broadcast_to(p['b2'], (1, EMB)),
    )
    out128 = _tc_final(h[0], h[1], u, seq, global_f, pvals)
    return out128[:, :1]


# speedup vs baseline: 7.8485x; 1.7467x over previous
import jax
import jax.numpy as jnp
from jax import lax
from jax.experimental import pallas as pl
from jax.experimental.pallas import tpu as pltpu
from jax.experimental.pallas import tpu_sc as plsc

N = 10000
E = 320000
G = 64
EMB = 128
HEADS = 4
NC = 2
NS = 16
FH = EMB // NC
CH = 128
EPT = E // NS
NCHUNK = -(-EPT // CH)
PAD = NCHUNK * CH - EPT
NBUF = 4
R8 = (N // NS) // 8 * 8
RREM = N - NS * R8
BN = 2000


def _leaky(v):
    return jnp.where(v > 0, v, 0.1 * v)


def _b16(v):
    return v.astype(jnp.bfloat16)


def _dot(a, b):
    return lax.dot_general(_b16(a), _b16(b), (((1,), (0,)), ((), ())),
                           preferred_element_type=jnp.float32)


def _sc_body(h_hbm, src_hbm, dst_hbm, zeros_hbm, agg_hbm,
             idx_src, idx_dst, rows,
             g0, g1, g2, g3, s0, s1, s2, s3, acc):
    gsem = (g0, g1, g2, g3)
    ssem = (s0, s1, s2, s3)
    c = lax.axis_index("c")
    s = lax.axis_index("s")
    r0 = s * R8
    pltpu.sync_copy(zeros_hbm.at[pl.ds(r0, R8)], acc.at[pl.ds(r0, R8)])

    @pl.when(s == NS - 1)
    def _zero_rem():
        pltpu.sync_copy(zeros_hbm.at[pl.ds(NS * R8, RREM)],
                        acc.at[pl.ds(NS * R8, RREM)])

    pltpu.sync_copy(src_hbm.at[c, s], idx_src)
    pltpu.sync_copy(dst_hbm.at[s], idx_dst)
    plsc.subcore_barrier()

    def start_gather(j, b):
        pltpu.async_copy(h_hbm.at[idx_src.at[j]], rows.at[b], gsem[b])

    def wait_gather(b):
        pltpu.make_async_copy(h_hbm.at[idx_src.at[0]], rows.at[b],
                              gsem[b]).wait()

    def start_scatter(j, b):
        pltpu.async_copy(rows.at[b], acc.at[idx_dst.at[j]], ssem[b], add=True)

    def wait_scatter(b):
        pltpu.make_async_copy(rows.at[b], acc.at[idx_dst.at[0]],
                              ssem[b]).wait()

    for b in range(NBUF):
        start_gather(b, b)

    def quad(m, carry):
        j0 = m * NBUF
        for b in range(NBUF):
            @pl.when(j0 + b < NCHUNK)
            def _scat(b=b):
                wait_gather(b)
                start_scatter(j0 + b, b)
        for b in range(NBUF):
            @pl.when(j0 + b + NBUF < NCHUNK)
            def _refill(b=b):
                wait_scatter(b)
                start_gather(j0 + b + NBUF, b)
        return carry

    lax.fori_loop(0, -(-NCHUNK // NBUF), quad, 0, unroll=False)
    for b in range(NBUF):
        wait_scatter(b)
    plsc.subcore_barrier()

    cN = c * N
    pltpu.sync_copy(acc.at[pl.ds(r0, R8)], agg_hbm.at[pl.ds(cN + r0, R8)])

    @pl.when(s == NS - 1)
    def _copy_rem():
        pltpu.sync_copy(acc.at[pl.ds(NS * R8, RREM)],
                        agg_hbm.at[pl.ds(cN + NS * R8, RREM)])


def _sc_spmm(h_flat, src4, dst3, zeros):
    mesh = plsc.VectorSubcoreMesh(core_axis_name="c", subcore_axis_name="s",
                                  num_cores=NC, num_subcores=NS)
    scratch = [
        pltpu.VMEM((NCHUNK, CH), jnp.int32),
        pltpu.VMEM((NCHUNK, CH), jnp.int32),
        pltpu.VMEM((NBUF, CH, FH), jnp.float32),
    ] + [pltpu.SemaphoreType.DMA] * (2 * NBUF) + [
        pltpu.VMEM_SHARED((N + 8, FH), jnp.float32),
    ]
    fn = pl.kernel(_sc_body,
                   out_type=jax.ShapeDtypeStruct((NC * N, FH), jnp.float32),
                   mesh=mesh, scratch_types=scratch,
                   compiler_params=pltpu.CompilerParams(use_tc_tiling_on_sc=False))
    return fn(h_flat, src4, dst3, zeros)


def _tc_layer_body(a0, a1, p0, p1, Wrel_r, Wroot_r, brel_r, out_ref):
    agg = jnp.concatenate([a0[...], a1[...]], axis=1)
    hp = jnp.concatenate([p0[...], p1[...]], axis=1)
    hn = _dot(agg, Wrel_r[...]) + brel_r[...] + _dot(hp, Wroot_r[...])
    out_ref[0] = hn[:, :FH]
    out_ref[1] = hn[:, FH:]


def _tc_layer(a0, a1, p0, p1, Wrel, Wroot, brel):
    def nmap(i):
        return (i, 0)

    def cmap(i):
        return (0, 0)

    half = pl.BlockSpec((BN, FH), nmap)
    return pl.pallas_call(
        _tc_layer_body,
        grid=(N // BN,),
        in_specs=[half, half, half, half,
                  pl.BlockSpec(Wrel.shape, cmap), pl.BlockSpec(Wroot.shape, cmap),
                  pl.BlockSpec((1, EMB), cmap)],
        out_specs=pl.BlockSpec((2, BN, FH), lambda i: (0, i, 0)),
        out_shape=jax.ShapeDtypeStruct((2, N, FH), jnp.float32),
        compiler_params=pltpu.CompilerParams(
            dimension_semantics=("arbitrary",)),
    )(a0, a1, p0, p1, Wrel, Wroot, brel)


def _tc_final_body(h30, h31, u_ref, seq_ref, gf_ref,
                   Wsr, bsr, Wgr, bgr,
                   Wqr, bqr, Wkr, bkr, Wvr, bvr, Wpr, bpr, scaler,
                   W1r, b1fr, W2r, b2fr,
                   out_ref, acc_ref):
    i = pl.program_id(0)

    @pl.when(i == 0)
    def _init():
        acc_ref[...] = jnp.zeros_like(acc_ref)

    ub = u_ref[...]
    dn = (((0,), (0,)), ((), ()))
    for k, r in enumerate((h30, h31)):
        acc_ref[k] += lax.dot_general(ub, r[...], dn,
                                      precision=lax.Precision.HIGHEST,
                                      preferred_element_type=jnp.float32)

    @pl.when(i == pl.num_programs(0) - 1)
    def _tail():
        g = jnp.concatenate([acc_ref[0], acc_ref[1]], axis=1)
        seq_rep = _leaky(_dot(seq_ref[...], Wsr[...]) + bsr[...])
        glob = _leaky(_dot(gf_ref[...], Wgr[...]) + bgr[...])
        toks = (g, seq_rep, glob)
        Q = [_dot(t, Wqr[...]) + bqr[...] for t in toks]
        K = [_dot(t, Wkr[...]) + bkr[...] for t in toks]
        V = [_dot(t, Wvr[...]) + bvr[...] for t in toks]
        scale = scaler[...]
        a1 = jnp.zeros((G, EMB), dtype=jnp.float32)
        for q in range(3):
            o_parts = []
            for h in range(HEADS):
                sl = slice(h * EMB, (h + 1) * EMB)
                Qb = _b16(Q[q][:, sl]).astype(jnp.float32)
                sc = [jnp.sum(Qb * _b16(K[kk][:, sl]).astype(jnp.float32),
                              axis=1, keepdims=True) / scale
                      for kk in range(3)]
                m = jnp.maximum(jnp.maximum(sc[0], sc[1]), sc[2])
                e = [jnp.exp(sv - m) for sv in sc]
                den = e[0] + e[1] + e[2]
                oh = jnp.zeros((G, EMB), dtype=jnp.float32)
                for kk in range(3):
                    ab = _b16(e[kk] / den).astype(jnp.float32)
                    vb = _b16(V[kk][:, sl]).astype(jnp.float32)
                    oh += ab * vb
                o_parts.append(oh)
            o_q = jnp.concatenate(o_parts, axis=1)
            a1 += _dot(o_q, Wpr[...]) + bpr[...]
        h1 = _leaky(_dot(a1, W1r[...]) + b1fr[...])
        out_ref[...] = _dot(h1, W2r[...]) + b2fr[...]


def _tc_final(h30, h31, u, seq, gf, pvals):
    def nmap(i):
        return (i, 0)

    def cmap(i):
        return (0, 0)

    half = pl.BlockSpec((BN, FH), nmap)
    in_specs = [half, half, pl.BlockSpec((BN, G), nmap)]
    in_specs += [pl.BlockSpec(t.shape, cmap) for t in (seq, gf)]
    in_specs += [pl.BlockSpec(t.shape, cmap) for t in pvals]
    return pl.pallas_call(
        _tc_final_body,
        grid=(N // BN,),
        in_specs=in_specs,
        out_specs=pl.BlockSpec((G, EMB), cmap),
        out_shape=jax.ShapeDtypeStruct((G, EMB), jnp.float32),
        scratch_shapes=[pltpu.VMEM((2, G, FH), jnp.float32)],
        compiler_params=pltpu.CompilerParams(
            dimension_semantics=("arbitrary",)),
    )(h30, h31, u, seq, gf, *pvals)


def kernel(x, edge_index, batch, seq, global_f, params):
    srcr = jnp.pad(edge_index[0].reshape(NS, EPT), ((0, 0), (0, PAD)),
                   constant_values=0)
    src4 = jnp.stack([srcr, srcr + N]).reshape(NC, NS, NCHUNK, CH)
    dst3 = jnp.pad(edge_index[1].reshape(NS, EPT), ((0, 0), (0, PAD)),
                   constant_values=N).reshape(NS, NCHUNK, CH)
    zeros = jnp.zeros((N, FH), dtype=jnp.float32)
    u = jax.nn.one_hot(batch, G, dtype=jnp.float32)
    p = params
    row = lambda a: a.reshape(1, -1)

    h = jnp.stack([x[:, :FH], x[:, FH:]])
    for l in range(3):
        agg_flat = _sc_spmm(h.reshape(NC * N, FH), src4, dst3, zeros)
        h = _tc_layer(agg_flat[:N], agg_flat[N:], h[0], h[1],
                      p['Wrel%d' % l], p['Wroot%d' % l], row(p['brel%d' % l]))

    pvals = (
        p['Ws'], row(p['bs']), p['Wg'], row(p['bg']),
        p['Wq'], row(p['bq']), p['Wk'], row(p['bk']),
        p['Wv'], row(p['bv']), p['Wp'], row(p['bp']),
        p['scale'].reshape(1, 1),
        p['W1'], row(p['b1']),
        jnp.broadcast_to(p['W2'], (EMB, EMB)), jnp.broadcast_to(p['b2'], (1, EMB)),
    )
    out128 = _tc_final(h[0], h[1], u, seq, global_f, pvals)
    return out128[:, :1]
